# baseline (speedup 1.0000x reference)
import jax
import jax.numpy as jnp
from jax.experimental import pallas as pl
from jax.experimental.pallas import tpu as pltpu


def _add_body(time_ref, x_ref, emb_ref, o_ref):
    e = emb_ref[0, 0, :]
    o_ref[0] = x_ref[0] + e[:, None]


def kernel(x, time, embeddings):
    b, d, h, w = x.shape
    hw = h * w
    xr = x.reshape(b, d, hw)
    t32 = time.astype(jnp.int32)
    emb3 = embeddings[:, :d].reshape(-1, 1, d)

    grid_spec = pltpu.PrefetchScalarGridSpec(
        num_scalar_prefetch=1,
        grid=(b,),
        in_specs=[
            pl.BlockSpec((1, d, hw), lambda i, t: (i, 0, 0)),
            pl.BlockSpec((1, 1, d), lambda i, t: (t[i], 0, 0)),
        ],
        out_specs=pl.BlockSpec((1, d, hw), lambda i, t: (i, 0, 0)),
    )
    out = pl.pallas_call(
        _add_body,
        grid_spec=grid_spec,
        out_shape=jax.ShapeDtypeStruct((b, d, hw), x.dtype),
    )(t32, xr, emb3)
    return out.reshape(b, d, h, w)
